# Initial kernel scaffold; baseline (speedup 1.0000x reference)
import jax
import jax.numpy as jnp
from jax.experimental import pallas as pl

_B_BLK = 16


def _diag_cov_kernel(x_ref, o_ref):
    d = jax.nn.softplus(x_ref[...])
    row = jax.lax.broadcasted_iota(jnp.int32, (1, 128, 128), 1)
    col = jax.lax.broadcasted_iota(jnp.int32, (1, 128, 128), 2)
    mask = row == col
    o_ref[...] = jnp.where(mask, d[:, :, None], jnp.float32(0.0))


def kernel(embeddings):
    batch, dim, _ = embeddings.shape
    x = embeddings[:, :, 0]
    grid = (batch // _B_BLK,)
    return pl.pallas_call(
        _diag_cov_kernel,
        grid=grid,
        in_specs=[pl.BlockSpec((_B_BLK, dim), lambda i: (i, 0))],
        out_specs=pl.BlockSpec((_B_BLK, dim, dim), lambda i: (i, 0, 0)),
        out_shape=jax.ShapeDtypeStruct((batch, dim, dim), embeddings.dtype),
    )(x)


# speedup vs baseline: 1.8503x; 1.8503x over previous
import jax
import jax.numpy as jnp
from jax.experimental import pallas as pl

_B_BLK = 16


def _diag_cov_kernel(x_ref, o_ref):
    d = jax.nn.softplus(x_ref[...])
    dim = d.shape[1]
    row = jax.lax.broadcasted_iota(jnp.int32, (dim, dim), 0)
    col = jax.lax.broadcasted_iota(jnp.int32, (dim, dim), 1)
    mask = row == col
    for b in range(d.shape[0]):
        o_ref[b, :, :] = jnp.where(mask, d[b][None, :], jnp.float32(0.0))


def kernel(embeddings):
    batch, dim, _ = embeddings.shape
    x = embeddings[:, :, 0]
    grid = (batch // _B_BLK,)
    return pl.pallas_call(
        _diag_cov_kernel,
        grid=grid,
        in_specs=[pl.BlockSpec((_B_BLK, dim), lambda i: (i, 0))],
        out_specs=pl.BlockSpec((_B_BLK, dim, dim), lambda i: (i, 0, 0)),
        out_shape=jax.ShapeDtypeStruct((batch, dim, dim), embeddings.dtype),
    )(x)


# speedup vs baseline: 2.8157x; 1.5217x over previous
import jax
import jax.numpy as jnp
from jax.experimental import pallas as pl

_B_BLK = 32


def _diag_cov_kernel(x_ref, o_ref):
    d = jax.nn.softplus(x_ref[...])
    dim = d.shape[1]
    row = jax.lax.broadcasted_iota(jnp.int32, (dim, dim), 0)
    col = jax.lax.broadcasted_iota(jnp.int32, (dim, dim), 1)
    mask = row == col
    for b in range(d.shape[0]):
        o_ref[b, :, :] = jnp.where(mask, d[b][None, :], jnp.float32(0.0))


def kernel(embeddings):
    batch, dim, _ = embeddings.shape
    x = embeddings[:, :, 0]
    grid = (batch // _B_BLK,)
    return pl.pallas_call(
        _diag_cov_kernel,
        grid=grid,
        in_specs=[pl.BlockSpec((_B_BLK, dim), lambda i: (i, 0))],
        out_specs=pl.BlockSpec((_B_BLK, dim, dim), lambda i: (i, 0, 0)),
        out_shape=jax.ShapeDtypeStruct((batch, dim, dim), embeddings.dtype),
    )(x)
